# baseline (speedup 1.0000x reference)
import functools
import math

import jax
import jax.numpy as jnp
from jax import lax
from jax.experimental import pallas as pl
from jax.experimental.pallas import tpu as pltpu
from jax.experimental.pallas import tpu_sc as plsc

_HALF_LOG_2PI = 0.5 * math.log(2.0 * math.pi)

_NW = 32
_LANES = 16


def _sc_stats(log_w):
    n = log_w.shape[0]
    per = n // _NW
    mesh = plsc.VectorSubcoreMesh(core_axis_name="c", subcore_axis_name="s")

    @functools.partial(
        pl.kernel,
        mesh=mesh,
        out_type=jax.ShapeDtypeStruct((_NW, _LANES), jnp.float32),
        scratch_types=[pltpu.VMEM((per,), jnp.float32),
                       pltpu.VMEM((_LANES,), jnp.float32)],
        compiler_params=pltpu.CompilerParams(needs_layout_passes=False),
    )
    def k(lw_hbm, out_hbm, buf, row):
        wid = lax.axis_index("s") * 2 + lax.axis_index("c")
        base = wid * per
        pltpu.sync_copy(lw_hbm.at[pl.ds(base, per)], buf)

        def max_step(j, mv):
            return jnp.maximum(mv, buf[pl.ds(j * _LANES, _LANES)])

        mv = lax.fori_loop(0, per // _LANES,  max_step,
                           jnp.full((_LANES,), -jnp.inf, jnp.float32))
        m_loc = jnp.max(mv)

        def sum_step(j, carry):
            s1v, s2v = carry
            e = jnp.exp(buf[pl.ds(j * _LANES, _LANES)] - m_loc)
            return s1v + e, s2v + e * e

        zeros = jnp.zeros((_LANES,), jnp.float32)
        s1v, s2v = lax.fori_loop(0, per // _LANES, sum_step, (zeros, zeros))
        s1 = jnp.sum(s1v)
        s2 = jnp.sum(s2v)
        idx = lax.broadcasted_iota(jnp.int32, (_LANES,), 0)
        vec = jnp.where(idx == 0, m_loc,
                        jnp.where(idx == 1, s1,
                                  jnp.where(idx == 2, s2, 0.0)))
        row[...] = vec
        pltpu.sync_copy(row, out_hbm.at[wid])

    return k(log_w)


def _combine_body(x_ref, out_ref):
    x = x_ref[...]
    ml = x[:, 0]
    m = jnp.max(ml)
    e = jnp.exp(ml - m)
    out_ref[0] = m
    out_ref[1] = jnp.sum(e * x[:, 1])
    out_ref[2] = jnp.sum(e * e * x[:, 2])


def _log_weight_stats(log_w):
    part = _sc_stats(log_w)
    return pl.pallas_call(
        _combine_body,
        out_shape=jax.ShapeDtypeStruct((3,), jnp.float32),
        out_specs=pl.BlockSpec(memory_space=pltpu.SMEM),
    )(part)


def _dense_body(d, p_ref, nz_ref, lw_ref, ab_ref, cb_ref, obs_ref,
                sc_ref, olw_ref, op_ref):
    pack = 128 // d
    b = p_ref.shape[0] // 128
    p = p_ref[...].reshape(b, 128)
    nz = nz_ref[...].reshape(b, 128)
    mean = jnp.dot(p, ab_ref[...].reshape(128, 128),
                   preferred_element_type=jnp.float32)
    nxt = mean + sc_ref[0] * nz
    om = jnp.dot(nxt, cb_ref[...].reshape(128, 128),
                 preferred_element_type=jnp.float32)
    z = (obs_ref[...].reshape(1, 128) - om) * sc_ref[1]
    lane = lax.broadcasted_iota(jnp.int32, (128, pack), 0)
    grp = lax.broadcasted_iota(jnp.int32, (128, pack), 1)
    sel = (lane // d == grp).astype(jnp.float32)
    emis = jnp.dot(z * z, sel, preferred_element_type=jnp.float32)
    olw_ref[...] = lw_ref[...] + (-0.5) * emis - sc_ref[2]
    op_ref[...] = nxt.reshape(b * 128)


def _dense(p_flat, nz_flat, lw2, A_big, C_big, obs_big, scalars,
           blk_particles, d):
    n = lw2.shape[0] * lw2.shape[1]
    pack = 128 // d
    total = p_flat.shape[0]
    blk = min(blk_particles, n)
    grid = (n // blk,)
    olw2, nxt = pl.pallas_call(
        functools.partial(_dense_body, d),
        grid=grid,
        in_specs=[
            pl.BlockSpec((blk * d,), lambda i: (i,)),
            pl.BlockSpec((blk * d,), lambda i: (i,)),
            pl.BlockSpec((blk // pack, pack), lambda i: (i, 0)),
            pl.BlockSpec((128 * 128,), lambda i: (0,)),
            pl.BlockSpec((128 * 128,), lambda i: (0,)),
            pl.BlockSpec((128,), lambda i: (0,)),
            pl.BlockSpec(memory_space=pltpu.SMEM),
        ],
        out_specs=[
            pl.BlockSpec((blk // pack, pack), lambda i: (i, 0)),
            pl.BlockSpec((blk * d,), lambda i: (i,)),
        ],
        out_shape=[
            jax.ShapeDtypeStruct((n // pack, pack), jnp.float32),
            jax.ShapeDtypeStruct((total,), jnp.float32),
        ],
    )(p_flat, nz_flat, lw2, A_big, C_big, obs_big, scalars)
    return olw2, nxt


def _cumsum_body(lw_ref, lse_ref, out_ref, carry):
    i = pl.program_id(0)

    @pl.when(i == 0)
    def _():
        carry[0] = 0.0

    w = jnp.exp(lw_ref[...] - lse_ref[0])
    b = w.shape[0]
    r = lax.broadcasted_iota(jnp.int32, (b, b), 0)
    c = lax.broadcasted_iota(jnp.int32, (b, b), 1)
    tri = (r >= c).astype(jnp.float32)
    cs = jnp.dot(tri, w, preferred_element_type=jnp.float32)
    c0 = carry[0]
    out_ref[...] = cs + c0
    carry[0] = c0 + jnp.sum(w)


def _cumsum(lw_col, lse):
    n = lw_col.shape[0]
    blk = 512
    return pl.pallas_call(
        _cumsum_body,
        grid=(n // blk,),
        in_specs=[
            pl.BlockSpec((blk, 1), lambda i: (i, 0)),
            pl.BlockSpec(memory_space=pltpu.SMEM),
        ],
        out_specs=pl.BlockSpec((blk, 1), lambda i: (i, 0)),
        out_shape=jax.ShapeDtypeStruct((n, 1), jnp.float32),
        scratch_shapes=[pltpu.SMEM((1,), jnp.float32)],
    )(lw_col, lse.reshape(1))


def _gather_body(n, u_ref, cw_ref, pv_ref, p_ref, out_ref):
    i = pl.program_id(0)
    j = pl.program_id(1)
    bo = out_ref.shape[0]
    bi = p_ref.shape[0]
    ii = i * bo + lax.broadcasted_iota(jnp.int32, (bo, 1), 0)
    pos = (u_ref[0] + ii.astype(jnp.float32)) / jnp.float32(n)
    cw = cw_ref[...]
    pv = pv_ref[...]
    jj = j * bi + lax.broadcasted_iota(jnp.int32, (1, bi), 1)
    sel = (pv < pos) & ((pos <= cw) | (jj == n - 1))
    contrib = jnp.dot(sel.astype(jnp.float32), p_ref[...],
                      preferred_element_type=jnp.float32)

    @pl.when(j == 0)
    def _():
        out_ref[...] = jnp.zeros_like(out_ref)

    out_ref[...] += contrib


def _systematic_gather(resample_u, cw_row, pv_row, particles):
    n, d = particles.shape
    bo, bi = 256, 512
    return pl.pallas_call(
        functools.partial(_gather_body, n),
        grid=(n // bo, n // bi),
        in_specs=[
            pl.BlockSpec(memory_space=pltpu.SMEM),
            pl.BlockSpec((1, bi), lambda i, j: (0, j)),
            pl.BlockSpec((1, bi), lambda i, j: (0, j)),
            pl.BlockSpec((bi, d), lambda i, j: (j, 0)),
        ],
        out_specs=pl.BlockSpec((bo, d), lambda i, j: (i, 0)),
        out_shape=jax.ShapeDtypeStruct((n, d), jnp.float32),
    )(resample_u, cw_row, pv_row, particles)


def kernel(log_w, particles, observation, A, C, log_sigma_x, log_sigma_y,
           resample_u, proposal_noise):
    n, d = particles.shape
    obs_dim = observation.shape[0]
    pack = 128 // d

    stats = _log_weight_stats(log_w)
    m, s1, s2 = stats[0], stats[1], stats[2]
    ess_e = (s1 * s1) / (s2 * n)

    lsy = log_sigma_y[0]
    scalars = jnp.stack([
        jnp.exp(log_sigma_x[0]),
        jnp.exp(-lsy),
        obs_dim * (lsy + _HALF_LOG_2PI),
    ])
    eye_p = jnp.eye(pack, dtype=jnp.float32)
    A_big = jnp.kron(eye_p, A.T).reshape(128 * 128)
    C_big = jnp.kron(eye_p, C.T).reshape(128 * 128)
    obs_big = jnp.tile(observation, pack)

    p_flat = particles.reshape(n * d)
    nz_flat = proposal_noise.reshape(n * d)
    lw2 = log_w.reshape(n // pack, pack)
    blk_particles = 32768

    def _hot(_):
        return _dense(p_flat, nz_flat, lw2, A_big, C_big, obs_big,
                      scalars, blk_particles, d)

    def _cold(_):
        lse = m + jnp.log(s1)
        cumw = _cumsum(log_w.reshape(n, 1), lse)
        cw_row = cumw.reshape(1, n)
        pv_row = jnp.concatenate(
            [jnp.full((1, 1), -jnp.inf, jnp.float32), cw_row[:, :-1]], axis=1)
        gathered = _systematic_gather(resample_u, cw_row, pv_row, particles)
        return _dense(gathered.reshape(n * d), nz_flat, jnp.zeros_like(lw2),
                      A_big, C_big, obs_big, scalars, blk_particles, d)

    out_lw, out_p = lax.cond(ess_e < 0.5, _cold, _hot, None)
    return out_lw.reshape(n), out_p.reshape(n, d), ess_e


# speedup vs baseline: 1.1785x; 1.1785x over previous
import functools
import math

import jax
import jax.numpy as jnp
from jax import lax
from jax.experimental import pallas as pl
from jax.experimental.pallas import tpu as pltpu
from jax.experimental.pallas import tpu_sc as plsc

_HALF_LOG_2PI = 0.5 * math.log(2.0 * math.pi)

_NW = 32
_LANES = 16


def _sc_stats(log_w):
    n = log_w.shape[0]
    per = n // _NW
    mesh = plsc.VectorSubcoreMesh(core_axis_name="c", subcore_axis_name="s")

    @functools.partial(
        pl.kernel,
        mesh=mesh,
        out_type=jax.ShapeDtypeStruct((_NW, _LANES), jnp.float32),
        scratch_types=[pltpu.VMEM((per,), jnp.float32),
                       pltpu.VMEM((_LANES,), jnp.float32)],
        compiler_params=pltpu.CompilerParams(needs_layout_passes=False),
    )
    def k(lw_hbm, out_hbm, buf, row):
        wid = lax.axis_index("s") * 2 + lax.axis_index("c")
        base = wid * per
        pltpu.sync_copy(lw_hbm.at[pl.ds(base, per)], buf)

        def max_step(j, mv):
            return jnp.maximum(mv, buf[pl.ds(j * _LANES, _LANES)])

        mv = lax.fori_loop(0, per // _LANES,  max_step,
                           jnp.full((_LANES,), -jnp.inf, jnp.float32))
        m_loc = jnp.max(mv)

        def sum_step(j, carry):
            s1v, s2v = carry
            e = jnp.exp(buf[pl.ds(j * _LANES, _LANES)] - m_loc)
            return s1v + e, s2v + e * e

        zeros = jnp.zeros((_LANES,), jnp.float32)
        s1v, s2v = lax.fori_loop(0, per // _LANES, sum_step, (zeros, zeros))
        s1 = jnp.sum(s1v)
        s2 = jnp.sum(s2v)
        idx = lax.broadcasted_iota(jnp.int32, (_LANES,), 0)
        vec = jnp.where(idx == 0, m_loc,
                        jnp.where(idx == 1, s1,
                                  jnp.where(idx == 2, s2, 0.0)))
        row[...] = vec
        pltpu.sync_copy(row, out_hbm.at[wid])

    return k(log_w)


def _combine_body(x_ref, out_ref):
    x = x_ref[...]
    ml = x[:, 0]
    m = jnp.max(ml)
    e = jnp.exp(ml - m)
    out_ref[0] = m
    out_ref[1] = jnp.sum(e * x[:, 1])
    out_ref[2] = jnp.sum(e * e * x[:, 2])


def _log_weight_stats(log_w):
    part = _sc_stats(log_w)
    return pl.pallas_call(
        _combine_body,
        out_shape=jax.ShapeDtypeStruct((3,), jnp.float32),
        out_specs=pl.BlockSpec(memory_space=pltpu.SMEM),
    )(part)


def _dense_body(d, p_ref, nz_ref, lw_ref, ab_ref, cb_ref, obs_ref,
                sc_ref, olw_ref, op_ref):
    pack = 128 // d
    b = p_ref.shape[0] // 128
    p = p_ref[...].reshape(b, 128)
    nz = nz_ref[...].reshape(b, 128)
    mean = jnp.dot(p, ab_ref[...].reshape(128, 128),
                   preferred_element_type=jnp.float32)
    nxt = mean + sc_ref[0] * nz
    om = jnp.dot(nxt, cb_ref[...].reshape(128, 128),
                 preferred_element_type=jnp.float32)
    z = (obs_ref[...].reshape(1, 128) - om) * sc_ref[1]
    lane = lax.broadcasted_iota(jnp.int32, (128, pack), 0)
    grp = lax.broadcasted_iota(jnp.int32, (128, pack), 1)
    sel = (lane // d == grp).astype(jnp.float32)
    emis = jnp.dot(z * z, sel, preferred_element_type=jnp.float32)
    olw_ref[...] = lw_ref[...] + (-0.5) * emis - sc_ref[2]
    op_ref[...] = nxt.reshape(b * 128)


def _dense(p_flat, nz_flat, lw2, A_big, C_big, obs_big, scalars,
           blk_particles, d):
    n = lw2.shape[0] * lw2.shape[1]
    pack = 128 // d
    total = p_flat.shape[0]
    blk = min(blk_particles, n)
    grid = (n // blk,)
    olw2, nxt = pl.pallas_call(
        functools.partial(_dense_body, d),
        grid=grid,
        in_specs=[
            pl.BlockSpec((blk * d,), lambda i: (i,)),
            pl.BlockSpec((blk * d,), lambda i: (i,)),
            pl.BlockSpec((blk // pack, pack), lambda i: (i, 0)),
            pl.BlockSpec((128 * 128,), lambda i: (0,)),
            pl.BlockSpec((128 * 128,), lambda i: (0,)),
            pl.BlockSpec((128,), lambda i: (0,)),
            pl.BlockSpec(memory_space=pltpu.SMEM),
        ],
        out_specs=[
            pl.BlockSpec((blk // pack, pack), lambda i: (i, 0)),
            pl.BlockSpec((blk * d,), lambda i: (i,)),
        ],
        out_shape=[
            jax.ShapeDtypeStruct((n // pack, pack), jnp.float32),
            jax.ShapeDtypeStruct((total,), jnp.float32),
        ],
    )(p_flat, nz_flat, lw2, A_big, C_big, obs_big, scalars)
    return olw2, nxt


def _cumsum_body(lw_ref, lse_ref, out_ref, carry):
    i = pl.program_id(0)

    @pl.when(i == 0)
    def _():
        carry[0] = 0.0

    w = jnp.exp(lw_ref[...] - lse_ref[0])
    b = w.shape[0]
    r = lax.broadcasted_iota(jnp.int32, (b, b), 0)
    c = lax.broadcasted_iota(jnp.int32, (b, b), 1)
    tri = (r >= c).astype(jnp.float32)
    cs = jnp.dot(tri, w, preferred_element_type=jnp.float32)
    c0 = carry[0]
    out_ref[...] = cs + c0
    carry[0] = c0 + jnp.sum(w)


def _cumsum(lw_col, lse):
    n = lw_col.shape[0]
    blk = 512
    return pl.pallas_call(
        _cumsum_body,
        grid=(n // blk,),
        in_specs=[
            pl.BlockSpec((blk, 1), lambda i: (i, 0)),
            pl.BlockSpec(memory_space=pltpu.SMEM),
        ],
        out_specs=pl.BlockSpec((blk, 1), lambda i: (i, 0)),
        out_shape=jax.ShapeDtypeStruct((n, 1), jnp.float32),
        scratch_shapes=[pltpu.SMEM((1,), jnp.float32)],
    )(lw_col, lse.reshape(1))


def _gather_body(n, u_ref, cw_ref, pv_ref, p_ref, out_ref):
    i = pl.program_id(0)
    j = pl.program_id(1)
    bo = out_ref.shape[0]
    bi = p_ref.shape[0]
    ii = i * bo + lax.broadcasted_iota(jnp.int32, (bo, 1), 0)
    pos = (u_ref[0] + ii.astype(jnp.float32)) / jnp.float32(n)
    cw = cw_ref[...]
    pv = pv_ref[...]
    jj = j * bi + lax.broadcasted_iota(jnp.int32, (1, bi), 1)
    sel = (pv < pos) & ((pos <= cw) | (jj == n - 1))
    contrib = jnp.dot(sel.astype(jnp.float32), p_ref[...],
                      preferred_element_type=jnp.float32)

    @pl.when(j == 0)
    def _():
        out_ref[...] = jnp.zeros_like(out_ref)

    out_ref[...] += contrib


def _systematic_gather(resample_u, cw_row, pv_row, particles):
    n, d = particles.shape
    bo, bi = 256, 512
    return pl.pallas_call(
        functools.partial(_gather_body, n),
        grid=(n // bo, n // bi),
        in_specs=[
            pl.BlockSpec(memory_space=pltpu.SMEM),
            pl.BlockSpec((1, bi), lambda i, j: (0, j)),
            pl.BlockSpec((1, bi), lambda i, j: (0, j)),
            pl.BlockSpec((bi, d), lambda i, j: (j, 0)),
        ],
        out_specs=pl.BlockSpec((bo, d), lambda i, j: (i, 0)),
        out_shape=jax.ShapeDtypeStruct((n, d), jnp.float32),
    )(resample_u, cw_row, pv_row, particles)


def kernel(log_w, particles, observation, A, C, log_sigma_x, log_sigma_y,
           resample_u, proposal_noise):
    n, d = particles.shape
    obs_dim = observation.shape[0]
    pack = 128 // d

    stats = _log_weight_stats(log_w)
    m, s1, s2 = stats[0], stats[1], stats[2]
    ess_e = (s1 * s1) / (s2 * n)

    lsy = log_sigma_y[0]
    scalars = jnp.stack([
        jnp.exp(log_sigma_x[0]),
        jnp.exp(-lsy),
        obs_dim * (lsy + _HALF_LOG_2PI),
    ])
    eye_p = jnp.eye(pack, dtype=jnp.float32)
    A_big = jnp.kron(eye_p, A.T).reshape(128 * 128)
    C_big = jnp.kron(eye_p, C.T).reshape(128 * 128)
    obs_big = jnp.tile(observation, pack)

    p_flat = particles.reshape(n * d)
    nz_flat = proposal_noise.reshape(n * d)
    lw2 = log_w.reshape(n // pack, pack)
    blk_particles = 32768

    hot_lw, hot_p = _dense(p_flat, nz_flat, lw2, A_big, C_big, obs_big,
                           scalars, blk_particles, d)

    def _hot(_):
        return hot_lw, hot_p

    def _cold(_):
        lse = m + jnp.log(s1)
        cumw = _cumsum(log_w.reshape(n, 1), lse)
        cw_row = cumw.reshape(1, n)
        pv_row = jnp.concatenate(
            [jnp.full((1, 1), -jnp.inf, jnp.float32), cw_row[:, :-1]], axis=1)
        gathered = _systematic_gather(resample_u, cw_row, pv_row, particles)
        return _dense(gathered.reshape(n * d), nz_flat, jnp.zeros_like(lw2),
                      A_big, C_big, obs_big, scalars, blk_particles, d)

    out_lw, out_p = lax.cond(ess_e < 0.5, _cold, _hot, None)
    return out_lw.reshape(n), out_p.reshape(n, d), ess_e
